# baseline (speedup 1.0000x reference)
import functools

import jax
import jax.numpy as jnp
from jax import lax
from jax.experimental import pallas as pl
from jax.experimental.pallas import tpu as pltpu
from jax.experimental.pallas import tpu_sc as plsc

_K = 8192
_D = 256
_B = 8
_HW = 1024
_T = _B * _HW
_CODE_TILE = 512
_NSTEP = _K // _CODE_TILE

_NW = 32
_BPW = _T // _NW


def _argmin_body(z_ref, cb_ref, idx_ref, zn_ref, minv_ref, mini_ref):
    j = pl.program_id(1)
    zb = z_ref[0]

    @pl.when(j == 0)
    def _init():
        t = zb * zb
        n = _D
        while n > 1:
            n //= 2
            t = t[:n] + t[n:]
        zn_ref[...] = t
        minv_ref[...] = jnp.full((1, _HW), jnp.inf, jnp.float32)
        mini_ref[...] = jnp.zeros((1, _HW), jnp.int32)

    cb = cb_ref[...]
    cn = jnp.sum(cb * cb, axis=1, keepdims=True)
    s = lax.dot_general(
        cb.astype(jnp.bfloat16), zb.astype(jnp.bfloat16),
        (((1,), (0,)), ((), ())),
        preferred_element_type=jnp.float32)
    d = (zn_ref[...] + cn) - 2.0 * s

    m = jnp.min(d, axis=0, keepdims=True)
    rows = lax.broadcasted_iota(jnp.int32, (_CODE_TILE, _HW), 0) + j * _CODE_TILE
    cand = jnp.min(jnp.where(d == m, rows, jnp.int32(2**30)),
                   axis=0, keepdims=True)
    better = m < minv_ref[...]
    mini_ref[...] = jnp.where(better, cand, mini_ref[...])
    minv_ref[...] = jnp.where(better, m, minv_ref[...])

    @pl.when(j == _NSTEP - 1)
    def _flush():
        idx_ref[...] = mini_ref[...][None]


def _compute_indices(z3, codebook):
    out = pl.pallas_call(
        _argmin_body,
        grid=(_B, _NSTEP),
        in_specs=[
            pl.BlockSpec((1, _D, _HW), lambda b, j: (b, 0, 0)),
            pl.BlockSpec((_CODE_TILE, _D), lambda b, j: (j, 0)),
        ],
        out_specs=pl.BlockSpec((1, 1, _HW), lambda b, j: (b, 0, 0)),
        out_shape=jax.ShapeDtypeStruct((_B, 1, _HW), jnp.int32),
        scratch_shapes=[
            pltpu.VMEM((1, _HW), jnp.float32),
            pltpu.VMEM((1, _HW), jnp.float32),
            pltpu.VMEM((1, _HW), jnp.int32),
        ],
        compiler_params=pltpu.CompilerParams(
            dimension_semantics=("arbitrary", "arbitrary")),
    )(z3, codebook)
    return out.reshape(_T)


def _sc_body(cb_hbm, idx_hbm, q_hbm, hist_hbm, idx_v, rows_v, cnt_v, sem):
    wid = lax.axis_index("s") * 2 + lax.axis_index("c")
    base = wid * _BPW
    pltpu.sync_copy(idx_hbm.at[pl.ds(base, _BPW)], idx_v)
    pltpu.async_copy(cb_hbm.at[idx_v], rows_v, sem).wait()
    pltpu.sync_copy(rows_v, q_hbm.at[pl.ds(base, _BPW)])

    zeros16 = jnp.zeros((16,), jnp.int32)

    def _zero(i, c):
        cnt_v[pl.ds(i * 16, 16)] = zeros16
        return c

    lax.fori_loop(0, _K // 16, _zero, 0)

    ones16 = jnp.ones((16,), jnp.int32)

    def _acc(i, c):
        iv = idx_v[pl.ds(i * 16, 16)]
        plsc.addupdate_scatter(cnt_v, [iv], ones16)
        return c

    lax.fori_loop(0, _BPW // 16, _acc, 0)
    pltpu.sync_copy(cnt_v, hist_hbm.at[wid])


@functools.cache
def _make_sc_gather():
    return pl.kernel(
        _sc_body,
        mesh=plsc.VectorSubcoreMesh(core_axis_name="c", subcore_axis_name="s"),
        out_type=[
            jax.ShapeDtypeStruct((_T, _D), jnp.float32),
            jax.ShapeDtypeStruct((_NW, _K), jnp.int32),
        ],
        scratch_types=[
            pltpu.VMEM((_BPW,), jnp.int32),
            pltpu.VMEM((_BPW, _D), jnp.float32),
            pltpu.VMEM((_K,), jnp.int32),
            pltpu.SemaphoreType.DMA,
        ],
        compiler_params=pltpu.CompilerParams(needs_layout_passes=False),
    )


def _sc_gather(codebook, indices):
    return _make_sc_gather()(codebook, indices)


def _final_body(z_ref, q_ref, hist_ref, qst_ref, loss_ref, perp_ref, acc_ref):
    b = pl.program_id(0)
    zb = z_ref[0]
    qT = q_ref[0].T
    qst_ref[0] = zb + (qT - zb)
    diff = zb - qT
    part = jnp.sum(diff * diff)

    @pl.when(b == 0)
    def _first():
        acc_ref[0, 0] = part

    @pl.when(b > 0)
    def _rest():
        acc_ref[0, 0] = acc_ref[0, 0] + part

    @pl.when(b == _B - 1)
    def _emit():
        loss_ref[...] = jnp.full((1, 1), acc_ref[0, 0] / jnp.float32(_T * _D),
                                 jnp.float32)
        counts = jnp.sum(hist_ref[...], axis=0)
        p = counts.astype(jnp.float32) * jnp.float32(1.0 / _T)
        logp = jnp.log(jnp.maximum(p, jnp.float32(1e-10)))
        ent = jnp.sum(p * logp)
        perp_ref[...] = jnp.full((1, 1), jnp.exp(-ent), jnp.float32)


def _finalize(z3, q_flat, hist):
    q3 = q_flat.reshape(_B, _HW, _D)
    qst, loss, perp = pl.pallas_call(
        _final_body,
        grid=(_B,),
        in_specs=[
            pl.BlockSpec((1, _D, _HW), lambda b: (b, 0, 0)),
            pl.BlockSpec((1, _HW, _D), lambda b: (b, 0, 0)),
            pl.BlockSpec((_NW, _K), lambda b: (0, 0)),
        ],
        out_specs=[
            pl.BlockSpec((1, _D, _HW), lambda b: (b, 0, 0)),
            pl.BlockSpec((1, 1), lambda b: (0, 0)),
            pl.BlockSpec((1, 1), lambda b: (0, 0)),
        ],
        out_shape=[
            jax.ShapeDtypeStruct((_B, _D, _HW), jnp.float32),
            jax.ShapeDtypeStruct((1, 1), jnp.float32),
            jax.ShapeDtypeStruct((1, 1), jnp.float32),
        ],
        scratch_shapes=[pltpu.SMEM((1, 1), jnp.float32)],
        compiler_params=pltpu.CompilerParams(
            dimension_semantics=("arbitrary",)),
    )(z3, q3, hist)
    return qst, loss, perp


def kernel(z, codebook):
    B, C, H, W = z.shape
    z3 = z.reshape(B, C, H * W)
    indices = _compute_indices(z3, codebook)
    q_flat, hist = _sc_gather(codebook, indices)
    qst3, loss, perp = _finalize(z3, q_flat, hist)
    quantized_st = qst3.reshape(B, C, H, W)
    loss_s = loss.reshape(())
    perp_s = perp.reshape(())
    return (quantized_st, indices, perp_s, loss_s, loss_s)
